# Initial kernel scaffold; baseline (speedup 1.0000x reference)
import jax
import jax.numpy as jnp
from jax import lax
from jax.experimental import pallas as pl
from jax.experimental.pallas import tpu as pltpu
from jax.experimental.pallas import tpu_sc as plsc

_NC = 2
_NS = 16
_NW = _NC * _NS
_L = 16


def _mm_bias_body(x_ref, w_ref, b_ref, o_ref):
    o_ref[...] = (
        jnp.dot(x_ref[...], w_ref[...], preferred_element_type=jnp.float32)
        + b_ref[...]
    )


def _linear(x, W, b, blk=None):
    M, D = x.shape
    H = W.shape[1]
    b2 = b.reshape(1, H).astype(jnp.float32)
    if blk is None:
        return pl.pallas_call(
            _mm_bias_body,
            out_shape=jax.ShapeDtypeStruct((M, H), jnp.float32),
        )(x, W, b2)
    assert M % blk == 0
    return pl.pallas_call(
        _mm_bias_body,
        grid=(M // blk,),
        in_specs=[
            pl.BlockSpec((blk, D), lambda i: (i, 0)),
            pl.BlockSpec((D, H), lambda i: (0, 0)),
            pl.BlockSpec((1, H), lambda i: (0, 0)),
        ],
        out_specs=pl.BlockSpec((blk, H), lambda i: (i, 0)),
        out_shape=jax.ShapeDtypeStruct((M, H), jnp.float32),
    )(x, W, b2)


def _final_body(p_ref, h_ref, wa_ref, ws_ref, b_ref, o_ref):
    agg = p_ref[0] + p_ref[1]
    o_ref[...] = jnp.maximum(
        jnp.dot(agg, wa_ref[...], preferred_element_type=jnp.float32)
        + jnp.dot(h_ref[...], ws_ref[...], preferred_element_type=jnp.float32)
        + b_ref[...],
        0.0,
    )


def _final(p, h, W_agg, W_self, b):
    N, H = h.shape
    return pl.pallas_call(
        _final_body,
        out_shape=jax.ShapeDtypeStruct((N, H), jnp.float32),
    )(p, h, W_agg, W_self, b.reshape(1, H).astype(jnp.float32))


def _sc_aggregate(h, src, dst, e, K=80):
    N, H = h.shape
    E = src.shape[0]
    assert E % (_NW * K) == 0
    C = E // (_NW * K)
    EW = C * K
    RPT = N // _NS
    ZR = 125
    assert N % _NS == 0 and RPT % ZR == 0

    mesh = plsc.VectorSubcoreMesh(
        core_axis_name="c", subcore_axis_name="s",
        num_cores=_NC, num_subcores=_NS,
    )

    def body(h_hbm, src_hbm, dst_hbm, e_hbm, out_hbm,
             sidx, didx, rows_v, e_v, zbuf, acc, sem):
        c = lax.axis_index("c")
        s = lax.axis_index("s")
        wid = s * _NC + c

        def zfill(i, _):
            zbuf[i // (H // _L), pl.ds((i % (H // _L)) * _L, _L)] = (
                jnp.zeros((_L,), jnp.float32))
            return 0
        lax.fori_loop(0, ZR * (H // _L), zfill, 0)

        def zcopy(k, _):
            pltpu.sync_copy(zbuf, acc.at[pl.ds(s * RPT + k * ZR, ZR)])
            return 0
        lax.fori_loop(0, RPT // ZR, zcopy, 0)
        plsc.subcore_barrier()

        base = wid * EW

        def step(i, _):
            off = base + i * K
            pltpu.sync_copy(src_hbm.at[pl.ds(off, K)], sidx)
            pltpu.sync_copy(dst_hbm.at[pl.ds(off, K)], didx)
            pltpu.async_copy(h_hbm.at[sidx], rows_v, sem).wait()
            pltpu.sync_copy(e_hbm.at[pl.ds(off, K)], e_v)

            def comp(t, _):
                r = t // (H // _L)
                sl = pl.ds((t % (H // _L)) * _L, _L)
                rows_v[r, sl] = jnp.maximum(rows_v[r, sl] + e_v[r, sl], 0.0)
                return 0
            lax.fori_loop(0, K * (H // _L), comp, 0)

            pltpu.sync_copy(rows_v, acc.at[didx], add=True)
            return 0
        lax.fori_loop(0, C, step, 0)
        plsc.subcore_barrier()

        pltpu.sync_copy(acc.at[pl.ds(s * RPT, RPT)],
                        out_hbm.at[c, pl.ds(s * RPT, RPT)])

    kern = pl.kernel(
        body,
        out_type=jax.ShapeDtypeStruct((_NC, N, H), jnp.float32),
        mesh=mesh,
        scratch_types=[
            pltpu.VMEM((K,), jnp.int32),
            pltpu.VMEM((K,), jnp.int32),
            pltpu.VMEM((K, H), jnp.float32),
            pltpu.VMEM((K, H), jnp.float32),
            pltpu.VMEM((ZR, H), jnp.float32),
            pltpu.VMEM_SHARED((N, H), jnp.float32),
            pltpu.SemaphoreType.DMA,
        ],
    )
    return kern(h, src, dst, e)


def kernel(x, edge_index, edge_attr,
           W_node, b_node, W_edge, b_edge, W_agg, W_self, b_gnn):
    h = _linear(x.astype(jnp.float32), W_node, b_node)
    e = _linear(edge_attr.astype(jnp.float32), W_edge, b_edge, blk=8000)
    src = edge_index[0]
    dst = edge_index[1]
    p = _sc_aggregate(h, src, dst, e)
    return _final(p, h, W_agg, W_self, b_gnn)


# speedup vs baseline: 1.6770x; 1.6770x over previous
import jax
import jax.numpy as jnp
from jax import lax
from jax.experimental import pallas as pl
from jax.experimental.pallas import tpu as pltpu
from jax.experimental.pallas import tpu_sc as plsc

_NC = 2
_NS = 16
_NW = _NC * _NS
_L = 16


def _mm_bias_body(x_ref, w_ref, b_ref, o_ref):
    o_ref[...] = (
        jnp.dot(x_ref[...], w_ref[...], preferred_element_type=jnp.float32)
        + b_ref[...]
    )


def _linear(x, W, b, blk=None):
    M, D = x.shape
    H = W.shape[1]
    b2 = b.reshape(1, H).astype(jnp.float32)
    if blk is None:
        return pl.pallas_call(
            _mm_bias_body,
            out_shape=jax.ShapeDtypeStruct((M, H), jnp.float32),
        )(x, W, b2)
    assert M % blk == 0
    return pl.pallas_call(
        _mm_bias_body,
        grid=(M // blk,),
        in_specs=[
            pl.BlockSpec((blk, D), lambda i: (i, 0)),
            pl.BlockSpec((D, H), lambda i: (0, 0)),
            pl.BlockSpec((1, H), lambda i: (0, 0)),
        ],
        out_specs=pl.BlockSpec((blk, H), lambda i: (i, 0)),
        out_shape=jax.ShapeDtypeStruct((M, H), jnp.float32),
    )(x, W, b2)


def _final_body(p_ref, h_ref, wa_ref, ws_ref, b_ref, o_ref):
    agg = p_ref[0] + p_ref[1]
    o_ref[...] = jnp.maximum(
        jnp.dot(agg, wa_ref[...], preferred_element_type=jnp.float32)
        + jnp.dot(h_ref[...], ws_ref[...], preferred_element_type=jnp.float32)
        + b_ref[...],
        0.0,
    )


def _final(p, h, W_agg, W_self, b):
    N, H = h.shape
    return pl.pallas_call(
        _final_body,
        out_shape=jax.ShapeDtypeStruct((N, H), jnp.float32),
    )(p, h, W_agg, W_self, b.reshape(1, H).astype(jnp.float32))


def _sc_aggregate(h, src, dst, e, K=80):
    N, H = h.shape
    E = src.shape[0]
    assert E % (_NW * K) == 0
    C = E // (_NW * K)
    EW = C * K
    DT = 10
    RPT = N // DT
    ZR = 200
    assert N % DT == 0 and RPT % ZR == 0 and RPT % 8 == 0 and ZR % 8 == 0

    mesh = plsc.VectorSubcoreMesh(
        core_axis_name="c", subcore_axis_name="s",
        num_cores=_NC, num_subcores=_NS,
    )

    def body(h_hbm, src_hbm, dst_hbm, e_hbm, out_hbm,
             sidx, didx, rows_v, e_v, zbuf, acc, sem):
        c = lax.axis_index("c")
        s = lax.axis_index("s")
        wid = s * _NC + c

        @pl.when(s < DT)
        def _zero():
            def zfill(i, _):
                zbuf[i // (H // _L), pl.ds((i % (H // _L)) * _L, _L)] = (
                    jnp.zeros((_L,), jnp.float32))
                return 0
            lax.fori_loop(0, ZR * (H // _L), zfill, 0)

            def zcopy(k, _):
                pltpu.sync_copy(zbuf, acc.at[pl.ds(s * RPT + k * ZR, ZR)])
                return 0
            lax.fori_loop(0, RPT // ZR, zcopy, 0)
        plsc.subcore_barrier()

        base = wid * EW

        def step(i, _):
            off = base + i * K
            pltpu.sync_copy(src_hbm.at[pl.ds(off, K)], sidx)
            pltpu.sync_copy(dst_hbm.at[pl.ds(off, K)], didx)
            pltpu.async_copy(h_hbm.at[sidx], rows_v, sem).wait()
            pltpu.sync_copy(e_hbm.at[pl.ds(off, K)], e_v)

            def comp(t, _):
                r = t // (H // _L)
                sl = pl.ds((t % (H // _L)) * _L, _L)
                rows_v[r, sl] = jnp.maximum(rows_v[r, sl] + e_v[r, sl], 0.0)
                return 0
            lax.fori_loop(0, K * (H // _L), comp, 0)

            pltpu.sync_copy(rows_v, acc.at[didx], add=True)
            return 0
        lax.fori_loop(0, C, step, 0)
        plsc.subcore_barrier()

        @pl.when(s < DT)
        def _dump():
            pltpu.sync_copy(acc.at[pl.ds(s * RPT, RPT)],
                            out_hbm.at[c, pl.ds(s * RPT, RPT)])

    kern = pl.kernel(
        body,
        out_type=jax.ShapeDtypeStruct((_NC, N, H), jnp.float32),
        mesh=mesh,
        scratch_types=[
            pltpu.VMEM((K,), jnp.int32),
            pltpu.VMEM((K,), jnp.int32),
            pltpu.VMEM((K, H), jnp.float32),
            pltpu.VMEM((K, H), jnp.float32),
            pltpu.VMEM((ZR, H), jnp.float32),
            pltpu.VMEM_SHARED((N, H), jnp.float32),
            pltpu.SemaphoreType.DMA,
        ],
    )
    return kern(h, src, dst, e)


def kernel(x, edge_index, edge_attr,
           W_node, b_node, W_edge, b_edge, W_agg, W_self, b_gnn):
    h = _linear(x.astype(jnp.float32), W_node, b_node)
    e = _linear(edge_attr.astype(jnp.float32), W_edge, b_edge, blk=8000)
    src = edge_index[0]
    dst = edge_index[1]
    p = _sc_aggregate(h, src, dst, e)
    return _final(p, h, W_agg, W_self, b_gnn)


# speedup vs baseline: 4.1708x; 2.4871x over previous
import jax
import jax.numpy as jnp
from jax import lax
from jax.experimental import pallas as pl
from jax.experimental.pallas import tpu as pltpu
from jax.experimental.pallas import tpu_sc as plsc

_NC = 2
_NS = 16
_NW = _NC * _NS
_L = 16


def _mm_bias_body(x_ref, w_ref, b_ref, o_ref):
    o_ref[...] = (
        jnp.dot(x_ref[...], w_ref[...], preferred_element_type=jnp.float32)
        + b_ref[...]
    )


def _linear(x, W, b, blk=None):
    M, D = x.shape
    H = W.shape[1]
    b2 = b.reshape(1, H).astype(jnp.float32)
    if blk is None:
        return pl.pallas_call(
            _mm_bias_body,
            out_shape=jax.ShapeDtypeStruct((M, H), jnp.float32),
        )(x, W, b2)
    assert M % blk == 0
    return pl.pallas_call(
        _mm_bias_body,
        grid=(M // blk,),
        in_specs=[
            pl.BlockSpec((blk, D), lambda i: (i, 0)),
            pl.BlockSpec((D, H), lambda i: (0, 0)),
            pl.BlockSpec((1, H), lambda i: (0, 0)),
        ],
        out_specs=pl.BlockSpec((blk, H), lambda i: (i, 0)),
        out_shape=jax.ShapeDtypeStruct((M, H), jnp.float32),
    )(x, W, b2)


def _final_body(p_ref, h_ref, wa_ref, ws_ref, b_ref, o_ref):
    agg = p_ref[0] + p_ref[1]
    o_ref[...] = jnp.maximum(
        jnp.dot(agg, wa_ref[...], preferred_element_type=jnp.float32)
        + jnp.dot(h_ref[...], ws_ref[...], preferred_element_type=jnp.float32)
        + b_ref[...],
        0.0,
    )


def _final(p, h, W_agg, W_self, b):
    N, H = h.shape
    return pl.pallas_call(
        _final_body,
        out_shape=jax.ShapeDtypeStruct((N, H), jnp.float32),
    )(p, h, W_agg, W_self, b.reshape(1, H).astype(jnp.float32))


def _sc_aggregate(h, src, dst, e, K=40):
    N, H = h.shape
    E = src.shape[0]
    assert E % (_NW * K) == 0
    C = E // (_NW * K)
    EW = C * K
    DT = 10
    RPT = N // DT
    ZR = 40
    assert N % DT == 0 and RPT % ZR == 0 and RPT % 8 == 0 and ZR % 8 == 0
    assert C % 2 == 0 and K % 8 == 0
    HV = H // _L

    mesh = plsc.VectorSubcoreMesh(
        core_axis_name="c", subcore_axis_name="s",
        num_cores=_NC, num_subcores=_NS,
    )

    def body(h_hbm, src_hbm, dst_hbm, e_hbm, out_hbm,
             sidx0, sidx1, didx0, didx1, rows0, rows1, e0, e1, zbuf, acc,
             sg0, sg1, se0, se1, ssi0, ssi1, sdi0, sdi1, sz):
        c = lax.axis_index("c")
        s = lax.axis_index("s")
        wid = s * _NC + c
        sidxs = (sidx0, sidx1)
        didxs = (didx0, didx1)
        rows = (rows0, rows1)
        ebufs = (e0, e1)
        sgs = (sg0, sg1)
        ses = (se0, se1)
        ssis = (ssi0, ssi1)
        sdis = (sdi0, sdi1)
        base = wid * EW

        @pl.when(s < DT)
        def _zero():
            def zfill(i, _):
                zbuf[i // HV, pl.ds((i % HV) * _L, _L)] = (
                    jnp.zeros((_L,), jnp.float32))
                return 0
            lax.fori_loop(0, ZR * HV, zfill, 0)

            def zfire(k, _):
                pltpu.async_copy(zbuf, acc.at[pl.ds(s * RPT + k * ZR, ZR)], sz)
                return 0
            lax.fori_loop(0, RPT // ZR, zfire, 0)

            def zdrain(k, _):
                pltpu.make_async_copy(
                    zbuf, acc.at[pl.ds(s * RPT, ZR)], sz).wait()
                return 0
            lax.fori_loop(0, RPT // ZR, zdrain, 0)

        def load_sidx(i, b):
            pltpu.async_copy(src_hbm.at[pl.ds(base + i * K, K)],
                             sidxs[b], ssis[b])

        def wait_sidx(i, b):
            pltpu.make_async_copy(src_hbm.at[pl.ds(base + i * K, K)],
                                  sidxs[b], ssis[b]).wait()

        def load_didx(i, b):
            pltpu.async_copy(dst_hbm.at[pl.ds(base + i * K, K)],
                             didxs[b], sdis[b])

        def wait_didx(i, b):
            pltpu.make_async_copy(dst_hbm.at[pl.ds(base + i * K, K)],
                                  didxs[b], sdis[b]).wait()

        def issue(i, b):
            pltpu.async_copy(h_hbm.at[sidxs[b]], rows[b], sgs[b])
            pltpu.async_copy(e_hbm.at[pl.ds(base + i * K, K)], ebufs[b], ses[b])

        def wait(i, b):
            pltpu.make_async_copy(h_hbm.at[sidxs[b]], rows[b], sgs[b]).wait()
            pltpu.make_async_copy(
                e_hbm.at[pl.ds(base + i * K, K)], ebufs[b], ses[b]).wait()

        def compute(b):
            rv, ev = rows[b], ebufs[b]

            @plsc.parallel_loop(0, K * HV, unroll=8)
            def _(t):
                r = t // HV
                sl = pl.ds((t % HV) * _L, _L)
                rv[r, sl] = jnp.maximum(rv[r, sl] + ev[r, sl], 0.0)

        load_sidx(0, 0)
        load_didx(0, 0)
        load_sidx(1, 1)
        load_didx(1, 1)
        wait_sidx(0, 0)
        issue(0, 0)
        wait_sidx(1, 1)
        issue(1, 1)
        plsc.subcore_barrier()

        def half(j, b):
            i = 2 * j + b
            wait(i, b)

            @pl.when(i + 2 < C)
            def _():
                load_sidx(i + 2, b)
            compute(b)
            wait_didx(i, b)
            pltpu.sync_copy(rows[b], acc.at[didxs[b]], add=True)

            @pl.when(i + 2 < C)
            def _():
                load_didx(i + 2, b)
                wait_sidx(i + 2, b)
                issue(i + 2, b)

        def pairbody(j, _):
            half(j, 0)
            half(j, 1)
            return 0
        lax.fori_loop(0, C // 2, pairbody, 0)
        plsc.subcore_barrier()

        @pl.when(s < DT)
        def _dump():
            pltpu.sync_copy(acc.at[pl.ds(s * RPT, RPT)],
                            out_hbm.at[c, pl.ds(s * RPT, RPT)])

    kern = pl.kernel(
        body,
        out_type=jax.ShapeDtypeStruct((_NC, N, H), jnp.float32),
        mesh=mesh,
        scratch_types=[
            pltpu.VMEM((K,), jnp.int32),
            pltpu.VMEM((K,), jnp.int32),
            pltpu.VMEM((K,), jnp.int32),
            pltpu.VMEM((K,), jnp.int32),
            pltpu.VMEM((K, H), jnp.float32),
            pltpu.VMEM((K, H), jnp.float32),
            pltpu.VMEM((K, H), jnp.float32),
            pltpu.VMEM((K, H), jnp.float32),
            pltpu.VMEM((ZR, H), jnp.float32),
            pltpu.VMEM_SHARED((N, H), jnp.float32),
            pltpu.SemaphoreType.DMA,
            pltpu.SemaphoreType.DMA,
            pltpu.SemaphoreType.DMA,
            pltpu.SemaphoreType.DMA,
            pltpu.SemaphoreType.DMA,
            pltpu.SemaphoreType.DMA,
            pltpu.SemaphoreType.DMA,
            pltpu.SemaphoreType.DMA,
            pltpu.SemaphoreType.DMA,
        ],
    )
    return kern(h, src, dst, e)


def kernel(x, edge_index, edge_attr,
           W_node, b_node, W_edge, b_edge, W_agg, W_self, b_gnn):
    h = _linear(x.astype(jnp.float32), W_node, b_node)
    e = _linear(edge_attr.astype(jnp.float32), W_edge, b_edge, blk=8000)
    p = _sc_aggregate(h, edge_index[0], edge_index[1], e)
    return _final(p, h, W_agg, W_self, b_gnn)


# speedup vs baseline: 4.2270x; 1.0135x over previous
import jax
import jax.numpy as jnp
from jax import lax
from jax.experimental import pallas as pl
from jax.experimental.pallas import tpu as pltpu
from jax.experimental.pallas import tpu_sc as plsc

_NC = 2
_NS = 16
_NW = _NC * _NS
_L = 16


def _mm_bias_body(x_ref, w_ref, b_ref, o_ref):
    o_ref[...] = (
        jnp.dot(x_ref[...], w_ref[...], preferred_element_type=jnp.float32)
        + b_ref[...]
    )


def _linear(x, W, b, blk=None):
    M, D = x.shape
    H = W.shape[1]
    b2 = b.reshape(1, H).astype(jnp.float32)
    if blk is None:
        return pl.pallas_call(
            _mm_bias_body,
            out_shape=jax.ShapeDtypeStruct((M, H), jnp.float32),
        )(x, W, b2)
    assert M % blk == 0
    return pl.pallas_call(
        _mm_bias_body,
        grid=(M // blk,),
        in_specs=[
            pl.BlockSpec((blk, D), lambda i: (i, 0)),
            pl.BlockSpec((D, H), lambda i: (0, 0)),
            pl.BlockSpec((1, H), lambda i: (0, 0)),
        ],
        out_specs=pl.BlockSpec((blk, H), lambda i: (i, 0)),
        out_shape=jax.ShapeDtypeStruct((M, H), jnp.float32),
    )(x, W, b2)


def _mm_pack_body(x_ref, w_ref, b_ref, o_ref):
    y = (jnp.dot(x_ref[...], w_ref[...], preferred_element_type=jnp.float32)
         + b_ref[...])
    yb = y.astype(jnp.bfloat16)
    lo = jax.lax.bitcast_convert_type(yb[:, :64], jnp.uint16).astype(jnp.uint32)
    hi = jax.lax.bitcast_convert_type(yb[:, 64:], jnp.uint16).astype(jnp.uint32)
    o_ref[...] = jax.lax.bitcast_convert_type(lo | (hi << 16), jnp.int32)


def _linear_pack(x, W, b, blk):
    M, D = x.shape
    H = W.shape[1]
    assert H % 2 == 0 and M % blk == 0
    b2 = b.reshape(1, H).astype(jnp.float32)
    return pl.pallas_call(
        _mm_pack_body,
        grid=(M // blk,),
        in_specs=[
            pl.BlockSpec((blk, D), lambda i: (i, 0)),
            pl.BlockSpec((D, H), lambda i: (0, 0)),
            pl.BlockSpec((1, H), lambda i: (0, 0)),
        ],
        out_specs=pl.BlockSpec((blk, H // 2), lambda i: (i, 0)),
        out_shape=jax.ShapeDtypeStruct((M, H // 2), jnp.int32),
    )(x, W, b2)


def _final_body(p_ref, h_ref, wa_ref, ws_ref, b_ref, o_ref):
    agg = p_ref[0] + p_ref[1]
    o_ref[...] = jnp.maximum(
        jnp.dot(agg, wa_ref[...], preferred_element_type=jnp.float32)
        + jnp.dot(h_ref[...], ws_ref[...], preferred_element_type=jnp.float32)
        + b_ref[...],
        0.0,
    )


def _final(p, h, W_agg, W_self, b):
    N, H = h.shape
    return pl.pallas_call(
        _final_body,
        out_shape=jax.ShapeDtypeStruct((N, H), jnp.float32),
    )(p, h, W_agg, W_self, b.reshape(1, H).astype(jnp.float32))


def _sc_aggregate(h, src, dst, e, K=40):
    N, H = h.shape
    E = src.shape[0]
    assert E % (_NW * K) == 0
    C = E // (_NW * K)
    EW = C * K
    DT = 10
    RPT = N // DT
    ZR = 40
    assert N % DT == 0 and RPT % ZR == 0 and RPT % 8 == 0 and ZR % 8 == 0
    assert C % 2 == 0 and K % 8 == 0
    HV = H // _L

    mesh = plsc.VectorSubcoreMesh(
        core_axis_name="c", subcore_axis_name="s",
        num_cores=_NC, num_subcores=_NS,
    )

    def body(h_hbm, src_hbm, dst_hbm, e_hbm, out_hbm,
             sidx0, sidx1, didx0, didx1, rows0, rows1, e0, e1, zbuf, acc,
             sg0, sg1, se0, se1, ssi0, ssi1, sdi0, sdi1, sz):
        c = lax.axis_index("c")
        s = lax.axis_index("s")
        wid = s * _NC + c
        sidxs = (sidx0, sidx1)
        didxs = (didx0, didx1)
        rows = (rows0, rows1)
        ebufs = (e0, e1)
        sgs = (sg0, sg1)
        ses = (se0, se1)
        ssis = (ssi0, ssi1)
        sdis = (sdi0, sdi1)
        base = wid * EW

        @pl.when(s < DT)
        def _zero():
            def zfill(i, _):
                zbuf[i // HV, pl.ds((i % HV) * _L, _L)] = (
                    jnp.zeros((_L,), jnp.float32))
                return 0
            lax.fori_loop(0, ZR * HV, zfill, 0)

            def zfire(k, _):
                pltpu.async_copy(zbuf, acc.at[pl.ds(s * RPT + k * ZR, ZR)], sz)
                return 0
            lax.fori_loop(0, RPT // ZR, zfire, 0)

            def zdrain(k, _):
                pltpu.make_async_copy(
                    zbuf, acc.at[pl.ds(s * RPT, ZR)], sz).wait()
                return 0
            lax.fori_loop(0, RPT // ZR, zdrain, 0)

        def load_sidx(i, b):
            pltpu.async_copy(src_hbm.at[pl.ds(base + i * K, K)],
                             sidxs[b], ssis[b])

        def wait_sidx(i, b):
            pltpu.make_async_copy(src_hbm.at[pl.ds(base + i * K, K)],
                                  sidxs[b], ssis[b]).wait()

        def load_didx(i, b):
            pltpu.async_copy(dst_hbm.at[pl.ds(base + i * K, K)],
                             didxs[b], sdis[b])

        def wait_didx(i, b):
            pltpu.make_async_copy(dst_hbm.at[pl.ds(base + i * K, K)],
                                  didxs[b], sdis[b]).wait()

        def issue(i, b):
            pltpu.async_copy(h_hbm.at[sidxs[b]], rows[b], sgs[b])
            pltpu.async_copy(e_hbm.at[pl.ds(base + i * K, K)], ebufs[b], ses[b])

        def wait(i, b):
            pltpu.make_async_copy(h_hbm.at[sidxs[b]], rows[b], sgs[b]).wait()
            pltpu.make_async_copy(
                e_hbm.at[pl.ds(base + i * K, K)], ebufs[b], ses[b]).wait()

        def compute(b):
            rv, ev = rows[b], ebufs[b]
            HW = HV // 2

            @plsc.parallel_loop(0, K * HW, unroll=4)
            def _(t):
                r = t // HW
                m = t % HW
                packed = ev[r, pl.ds(m * _L, _L)]
                elo = jax.lax.bitcast_convert_type(packed << 16, jnp.float32)
                ehi = jax.lax.bitcast_convert_type(
                    packed & jnp.int32(-65536), jnp.float32)
                sl_lo = pl.ds(m * _L, _L)
                sl_hi = pl.ds(HW * _L + m * _L, _L)
                rv[r, sl_lo] = jnp.maximum(rv[r, sl_lo] + elo, 0.0)
                rv[r, sl_hi] = jnp.maximum(rv[r, sl_hi] + ehi, 0.0)

        load_sidx(0, 0)
        load_didx(0, 0)
        load_sidx(1, 1)
        load_didx(1, 1)
        wait_sidx(0, 0)
        issue(0, 0)
        wait_sidx(1, 1)
        issue(1, 1)
        plsc.subcore_barrier()

        def half(j, b):
            i = 2 * j + b
            wait(i, b)

            @pl.when(i + 2 < C)
            def _():
                load_sidx(i + 2, b)
            compute(b)
            wait_didx(i, b)
            pltpu.sync_copy(rows[b], acc.at[didxs[b]], add=True)

            @pl.when(i + 2 < C)
            def _():
                load_didx(i + 2, b)
                wait_sidx(i + 2, b)
                issue(i + 2, b)

        def pairbody(j, _):
            half(j, 0)
            half(j, 1)
            return 0
        lax.fori_loop(0, C // 2, pairbody, 0)
        plsc.subcore_barrier()

        @pl.when(s < DT)
        def _dump():
            pltpu.sync_copy(acc.at[pl.ds(s * RPT, RPT)],
                            out_hbm.at[c, pl.ds(s * RPT, RPT)])

    kern = pl.kernel(
        body,
        out_type=jax.ShapeDtypeStruct((_NC, N, H), jnp.float32),
        mesh=mesh,
        scratch_types=[
            pltpu.VMEM((K,), jnp.int32),
            pltpu.VMEM((K,), jnp.int32),
            pltpu.VMEM((K,), jnp.int32),
            pltpu.VMEM((K,), jnp.int32),
            pltpu.VMEM((K, H), jnp.float32),
            pltpu.VMEM((K, H), jnp.float32),
            pltpu.VMEM((K, H // 2), jnp.int32),
            pltpu.VMEM((K, H // 2), jnp.int32),
            pltpu.VMEM((ZR, H), jnp.float32),
            pltpu.VMEM_SHARED((N, H), jnp.float32),
            pltpu.SemaphoreType.DMA,
            pltpu.SemaphoreType.DMA,
            pltpu.SemaphoreType.DMA,
            pltpu.SemaphoreType.DMA,
            pltpu.SemaphoreType.DMA,
            pltpu.SemaphoreType.DMA,
            pltpu.SemaphoreType.DMA,
            pltpu.SemaphoreType.DMA,
            pltpu.SemaphoreType.DMA,
        ],
    )
    return kern(h, src, dst, e)


def kernel(x, edge_index, edge_attr,
           W_node, b_node, W_edge, b_edge, W_agg, W_self, b_gnn):
    h = _linear(x.astype(jnp.float32), W_node, b_node)
    e = _linear_pack(edge_attr.astype(jnp.float32), W_edge, b_edge, blk=8000)
    p = _sc_aggregate(h, edge_index[0], edge_index[1], e)
    return _final(p, h, W_agg, W_self, b_gnn)


# speedup vs baseline: 4.2275x; 1.0001x over previous
import jax
import jax.numpy as jnp
from jax import lax
from jax.experimental import pallas as pl
from jax.experimental.pallas import tpu as pltpu
from jax.experimental.pallas import tpu_sc as plsc

_NC = 2
_NS = 16
_NW = _NC * _NS
_L = 16


def _mm_bias_body(x_ref, w_ref, b_ref, o_ref):
    o_ref[...] = (
        jnp.dot(x_ref[...], w_ref[...], preferred_element_type=jnp.float32)
        + b_ref[...]
    )


def _pack_bf16_pairs(y):
    Hh = y.shape[1] // 2
    yb = y.astype(jnp.bfloat16)
    lo = jax.lax.bitcast_convert_type(yb[:, :Hh], jnp.uint16).astype(jnp.uint32)
    hi = jax.lax.bitcast_convert_type(yb[:, Hh:], jnp.uint16).astype(jnp.uint32)
    return jax.lax.bitcast_convert_type(lo | (hi << 16), jnp.int32)


def _mm_bias_dual_body(x_ref, w_ref, b_ref, o_ref, o32_ref):
    y = (jnp.dot(x_ref[...], w_ref[...], preferred_element_type=jnp.float32)
         + b_ref[...])
    o_ref[...] = y
    o32_ref[...] = _pack_bf16_pairs(y)


def _linear_dual(x, W, b):
    M, D = x.shape
    H = W.shape[1]
    b2 = b.reshape(1, H).astype(jnp.float32)
    return pl.pallas_call(
        _mm_bias_dual_body,
        out_shape=[
            jax.ShapeDtypeStruct((M, H), jnp.float32),
            jax.ShapeDtypeStruct((M, H // 2), jnp.int32),
        ],
    )(x, W, b2)


def _linear(x, W, b, blk=None):
    M, D = x.shape
    H = W.shape[1]
    b2 = b.reshape(1, H).astype(jnp.float32)
    if blk is None:
        return pl.pallas_call(
            _mm_bias_body,
            out_shape=jax.ShapeDtypeStruct((M, H), jnp.float32),
        )(x, W, b2)
    assert M % blk == 0
    return pl.pallas_call(
        _mm_bias_body,
        grid=(M // blk,),
        in_specs=[
            pl.BlockSpec((blk, D), lambda i: (i, 0)),
            pl.BlockSpec((D, H), lambda i: (0, 0)),
            pl.BlockSpec((1, H), lambda i: (0, 0)),
        ],
        out_specs=pl.BlockSpec((blk, H), lambda i: (i, 0)),
        out_shape=jax.ShapeDtypeStruct((M, H), jnp.float32),
    )(x, W, b2)


def _mm_pack_body(x_ref, w_ref, b_ref, o_ref):
    y = (jnp.dot(x_ref[...], w_ref[...], preferred_element_type=jnp.float32)
         + b_ref[...])
    o_ref[...] = _pack_bf16_pairs(y)


def _linear_pack(x, W, b, blk):
    M, D = x.shape
    H = W.shape[1]
    assert H % 2 == 0 and M % blk == 0
    b2 = b.reshape(1, H).astype(jnp.float32)
    return pl.pallas_call(
        _mm_pack_body,
        grid=(M // blk,),
        in_specs=[
            pl.BlockSpec((blk, D), lambda i: (i, 0)),
            pl.BlockSpec((D, H), lambda i: (0, 0)),
            pl.BlockSpec((1, H), lambda i: (0, 0)),
        ],
        out_specs=pl.BlockSpec((blk, H // 2), lambda i: (i, 0)),
        out_shape=jax.ShapeDtypeStruct((M, H // 2), jnp.int32),
    )(x, W, b2)


def _final_body(p_ref, h_ref, wa_ref, ws_ref, b_ref, o_ref):
    agg = p_ref[0] + p_ref[1]
    o_ref[...] = jnp.maximum(
        jnp.dot(agg, wa_ref[...], preferred_element_type=jnp.float32)
        + jnp.dot(h_ref[...], ws_ref[...], preferred_element_type=jnp.float32)
        + b_ref[...],
        0.0,
    )


def _final(p, h, W_agg, W_self, b):
    N, H = h.shape
    return pl.pallas_call(
        _final_body,
        out_shape=jax.ShapeDtypeStruct((N, H), jnp.float32),
    )(p, h, W_agg, W_self, b.reshape(1, H).astype(jnp.float32))


def _sc_aggregate(h32, src, dst, e, H, K=40):
    N = h32.shape[0]
    E = src.shape[0]
    assert E % (_NW * K) == 0
    C = E // (_NW * K)
    EW = C * K
    DT = 10
    RPT = N // DT
    ZR = 40
    assert N % DT == 0 and RPT % ZR == 0 and RPT % 8 == 0 and ZR % 8 == 0
    assert C % 2 == 0 and K % 8 == 0
    HV = H // _L

    mesh = plsc.VectorSubcoreMesh(
        core_axis_name="c", subcore_axis_name="s",
        num_cores=_NC, num_subcores=_NS,
    )

    def body(h_hbm, src_hbm, dst_hbm, e_hbm, out_hbm,
             sidx0, sidx1, didx0, didx1, rows0, rows1, e0, e1, zbuf, acc,
             sg0, sg1, se0, se1, ssi0, ssi1, sdi0, sdi1, sz):
        c = lax.axis_index("c")
        s = lax.axis_index("s")
        wid = s * _NC + c
        sidxs = (sidx0, sidx1)
        didxs = (didx0, didx1)
        rows = (rows0, rows1)
        ebufs = (e0, e1)
        sgs = (sg0, sg1)
        ses = (se0, se1)
        ssis = (ssi0, ssi1)
        sdis = (sdi0, sdi1)
        base = wid * EW

        @pl.when(s < DT)
        def _zero():
            def zfill(i, _):
                zbuf[i // HV, pl.ds((i % HV) * _L, _L)] = (
                    jnp.zeros((_L,), jnp.float32))
                return 0
            lax.fori_loop(0, ZR * HV, zfill, 0)

            def zfire(k, _):
                pltpu.async_copy(zbuf, acc.at[pl.ds(s * RPT + k * ZR, ZR)], sz)
                return 0
            lax.fori_loop(0, RPT // ZR, zfire, 0)

            def zdrain(k, _):
                pltpu.make_async_copy(
                    zbuf, acc.at[pl.ds(s * RPT, ZR)], sz).wait()
                return 0
            lax.fori_loop(0, RPT // ZR, zdrain, 0)

        def load_sidx(i, b):
            pltpu.async_copy(src_hbm.at[pl.ds(base + i * K, K)],
                             sidxs[b], ssis[b])

        def wait_sidx(i, b):
            pltpu.make_async_copy(src_hbm.at[pl.ds(base + i * K, K)],
                                  sidxs[b], ssis[b]).wait()

        def load_didx(i, b):
            pltpu.async_copy(dst_hbm.at[pl.ds(base + i * K, K)],
                             didxs[b], sdis[b])

        def wait_didx(i, b):
            pltpu.make_async_copy(dst_hbm.at[pl.ds(base + i * K, K)],
                                  didxs[b], sdis[b]).wait()

        def issue(i, b):
            pltpu.async_copy(h_hbm.at[sidxs[b]], rows[b], sgs[b])
            pltpu.async_copy(e_hbm.at[pl.ds(base + i * K, K)], ebufs[b], ses[b])

        def wait(i, b):
            pltpu.make_async_copy(h_hbm.at[sidxs[b]], rows[b], sgs[b]).wait()
            pltpu.make_async_copy(
                e_hbm.at[pl.ds(base + i * K, K)], ebufs[b], ses[b]).wait()

        def compute(b):
            rv, ev = rows[b], ebufs[b]
            HW = HV // 2

            @plsc.parallel_loop(0, K * HW, unroll=4)
            def _(t):
                r = t // HW
                m = t % HW
                packed = ev[r, pl.ds(m * _L, _L)]
                elo = jax.lax.bitcast_convert_type(packed << 16, jnp.float32)
                ehi = jax.lax.bitcast_convert_type(
                    packed & jnp.int32(-65536), jnp.float32)
                sl_lo = pl.ds(m * _L, _L)
                sl_hi = pl.ds(HW * _L + m * _L, _L)
                rv[r, sl_lo] = jnp.maximum(rv[r, sl_lo] + elo, 0.0)
                rv[r, sl_hi] = jnp.maximum(rv[r, sl_hi] + ehi, 0.0)

        load_sidx(0, 0)
        load_didx(0, 0)
        load_sidx(1, 1)
        load_didx(1, 1)
        wait_sidx(0, 0)
        issue(0, 0)
        wait_sidx(1, 1)
        issue(1, 1)
        plsc.subcore_barrier()

        def half(j, b):
            i = 2 * j + b
            wait(i, b)

            @pl.when(i + 2 < C)
            def _():
                load_sidx(i + 2, b)
            compute(b)
            wait_didx(i, b)
            pltpu.sync_copy(rows[b], acc.at[didxs[b]], add=True)

            @pl.when(i + 2 < C)
            def _():
                load_didx(i + 2, b)
                wait_sidx(i + 2, b)
                issue(i + 2, b)

        def pairbody(j, _):
            half(j, 0)
            half(j, 1)
            return 0
        lax.fori_loop(0, C // 2, pairbody, 0)
        plsc.subcore_barrier()

        @pl.when(s < DT)
        def _dump():
            pltpu.sync_copy(acc.at[pl.ds(s * RPT, RPT)],
                            out_hbm.at[c, pl.ds(s * RPT, RPT)])

    kern = pl.kernel(
        body,
        out_type=jax.ShapeDtypeStruct((_NC, N, H), jnp.float32),
        mesh=mesh,
        scratch_types=[
            pltpu.VMEM((K,), jnp.int32),
            pltpu.VMEM((K,), jnp.int32),
            pltpu.VMEM((K,), jnp.int32),
            pltpu.VMEM((K,), jnp.int32),
            pltpu.VMEM((K, H), jnp.float32),
            pltpu.VMEM((K, H), jnp.float32),
            pltpu.VMEM((K, H // 2), jnp.int32),
            pltpu.VMEM((K, H // 2), jnp.int32),
            pltpu.VMEM((ZR, H), jnp.float32),
            pltpu.VMEM_SHARED((N, H), jnp.float32),
            pltpu.SemaphoreType.DMA,
            pltpu.SemaphoreType.DMA,
            pltpu.SemaphoreType.DMA,
            pltpu.SemaphoreType.DMA,
            pltpu.SemaphoreType.DMA,
            pltpu.SemaphoreType.DMA,
            pltpu.SemaphoreType.DMA,
            pltpu.SemaphoreType.DMA,
            pltpu.SemaphoreType.DMA,
        ],
    )
    return kern(h32, src, dst, e)


def kernel(x, edge_index, edge_attr,
           W_node, b_node, W_edge, b_edge, W_agg, W_self, b_gnn):
    h = _linear(x.astype(jnp.float32), W_node, b_node)
    e = _linear_pack(edge_attr.astype(jnp.float32), W_edge, b_edge, blk=8000)
    p = _sc_aggregate(h, edge_index[0], edge_index[1], e, H=h.shape[1])
    return _final(p, h, W_agg, W_self, b_gnn)
